# Initial kernel scaffold; baseline (speedup 1.0000x reference)
import jax
import jax.numpy as jnp
from jax.experimental import pallas as pl


def kernel(end_points, start_points, W0, b0, W1, b1, W2, b2, W3, b3):
    raise NotImplementedError("write your pallas kernel here")


# speedup vs baseline: 17.8747x; 17.8747x over previous
import jax
import jax.numpy as jnp
from jax.experimental import pallas as pl
from jax.experimental.pallas import tpu as pltpu

_B = 256
_D = 3
_H = 256
_K = 64
_EPS = 1e-06
_NC = 4
_R = 16
_KP = 384


_LAYER_PREC = jax.lax.Precision.DEFAULT


def _dot(a, b, prec=jax.lax.Precision.HIGHEST):
    return jax.lax.dot_general(
        a, b, (((1,), (0,)), ((), ())), precision=prec,
        preferred_element_type=jnp.float32)


def _dotb(a, b):
    return jax.lax.dot_general(
        a, b, (((2,), (1,)), ((0,), (0,))), precision=jax.lax.Precision.HIGHEST,
        preferred_element_type=jnp.float32)


def _insert(t, vm, z):
    R = t.shape[0]
    zL = z[:, :-1, :]
    dz = z[:, 1:, :] - zL
    segv = vm[:, :-1] * vm[:, 1:]
    dvalid = jnp.abs(dz) > _EPS
    dsafe = jnp.where(dvalid, dz, jnp.float32(1.0))
    a = -zL / dsafe
    cva = (segv[:, :, None] > 0) & dvalid & (a > _EPS) & (a < 1.0 - _EPS)
    BIGK = jnp.float32(10.0)
    key = jnp.where(cva, a, BIGK)
    lane = jax.lax.broadcasted_iota(jnp.int32, key.shape, 2)
    alphas = []
    mcnt = jnp.zeros((R, _K - 1), jnp.float32)
    for _ in range(_NC):
        m = jnp.min(key, axis=2, keepdims=True)
        is_min = key == m
        sel = jnp.min(jnp.where(is_min, lane, _H), axis=2, keepdims=True)
        key = jnp.where(lane == sel, BIGK, key)
        alphas.append(m[:, :, 0])
        mcnt = mcnt + (m[:, :, 0] < 9.0).astype(jnp.float32)

    pad1 = jnp.zeros((R, 1), jnp.float32)
    mpad = jnp.concatenate([mcnt, pad1], axis=1)
    lt = (jax.lax.broadcasted_iota(jnp.int32, (_K, _K), 0)
          < jax.lax.broadcasted_iota(jnp.int32, (_K, _K), 1)).astype(jnp.float32)
    Cm = _dot(mpad, lt)
    ivec = jax.lax.broadcasted_iota(jnp.int32, (R, _K), 1).astype(jnp.float32)
    BIGP = jnp.float32(1e9)
    p = jnp.where(vm > 0, ivec + Cm, BIGP)
    nnew = jnp.minimum(
        jnp.sum(vm, axis=1, keepdims=True) + jnp.sum(mcnt, axis=1, keepdims=True),
        jnp.float32(_K))

    jsub = jax.lax.broadcasted_iota(jnp.int32, (R, _K, 1), 1).astype(jnp.float32)
    le = (p[:, None, :] <= jsub).astype(jnp.float32)
    iLf = jnp.sum(le, axis=2, keepdims=True) - 1.0
    ilane = jax.lax.broadcasted_iota(jnp.int32, (R, _K, _K), 2).astype(jnp.float32)
    O = (ilane == iLf).astype(jnp.float32)
    posL = jnp.sum(O * p[:, None, :], axis=2, keepdims=True)
    cidx = jsub - posL - 1.0
    aout = jnp.zeros((R, _K, 1), jnp.float32)
    for c in range(_NC):
        acpad = jnp.concatenate([alphas[c], pad1], axis=1)
        gc = jnp.sum(O * acpad[:, None, :], axis=2, keepdims=True)
        aout = aout + jnp.where(cidx == jnp.float32(c), gc, jnp.float32(0.0))

    vnew = (jsub < nnew[:, :, None]).astype(jnp.float32)
    dtfull = jnp.concatenate([t[:, 1:] - t[:, :-1], pad1], axis=1)
    tL = jnp.sum(O * t[:, None, :], axis=2, keepdims=True)
    dtv = jnp.sum(O * dtfull[:, None, :], axis=2, keepdims=True)
    tnew = jnp.where(vnew > 0, tL + aout * dtv, jnp.float32(1.0))

    dzpad = jnp.concatenate([dz, jnp.zeros((R, 1, _H), jnp.float32)], axis=1)
    zcat = jnp.concatenate([z, dzpad], axis=2)
    g = _dotb(O, zcat)
    znew = g[:, :, :_H] + aout * g[:, :, _H:]
    znew = jnp.where(vnew > 0, znew, jnp.float32(0.0))
    return tnew[:, :, 0], vnew[:, :, 0], znew


def _body(ep_ref, sp_ref, w0t_ref, b0_ref, w1t_ref, b1_ref, w2t_ref, b2_ref,
          w3t_ref, b3_ref, t_out_ref, v_out_ref, y_out_ref):
    ep = ep_ref[...]
    sp = sp_ref[...]
    d = ep - sp
    R = ep.shape[0]

    t = jnp.where(jax.lax.broadcasted_iota(jnp.int32, (R, _K), 1) == 0,
                  jnp.float32(0.0), jnp.float32(1.0))
    vm = (jax.lax.broadcasted_iota(jnp.int32, (R, _K), 1) < 2).astype(jnp.float32)

    a0 = _dot(d, w0t_ref[...], _LAYER_PREC)
    beta0 = _dot(sp, w0t_ref[...], _LAYER_PREC) + b0_ref[...]
    z = a0[:, None, :] * t[:, :, None] + beta0[:, None, :]
    z = jnp.where(vm[:, :, None] > 0, z, jnp.float32(0.0))
    t, vm, z = _insert(t, vm, z)

    for wt_ref, b_ref in ((w1t_ref, b1_ref), (w2t_ref, b2_ref)):
        h = jnp.where(vm[:, :, None] > 0, jnp.maximum(z, 0.0), jnp.float32(0.0))
        coords = sp[:, None, :_D] + t[:, :, None] * d[:, None, :_D]
        coords = jnp.where(vm[:, :, None] > 0, coords, jnp.float32(0.0))
        cpad = jnp.pad(coords, ((0, 0), (0, 0), (0, _KP - _H - _D)))
        hin = jnp.concatenate([h, cpad], axis=2)
        z = _dot(hin.reshape(R * _K, _KP), wt_ref[...], _LAYER_PREC) + b_ref[...]
        z = z.reshape(R, _K, _H)
        z = jnp.where(vm[:, :, None] > 0, z, jnp.float32(0.0))
        t, vm, z = _insert(t, vm, z)

    h = jnp.where(vm[:, :, None] > 0, jnp.maximum(z, 0.0), jnp.float32(0.0))
    y = _dot(h.reshape(R * _K, _H), w3t_ref[...], _LAYER_PREC)
    y = (y + b3_ref[...]).reshape(R, _K, 128)[:, :, 0]
    y = jnp.where(vm > 0, y, jnp.float32(0.0))

    t_out_ref[...] = t
    v_out_ref[...] = vm
    y_out_ref[...] = y


def kernel(end_points, start_points, W0, b0, W1, b1, W2, b2, W3, b3):
    epp = jnp.pad(end_points, ((0, 0), (0, 128 - _D)))
    spp = jnp.pad(start_points, ((0, 0), (0, 128 - _D)))
    w0t = jnp.pad(W0.T, ((0, 128 - _D), (0, 0)))
    w1t = jnp.pad(W1.T, ((0, _KP - (_H + _D)), (0, 0)))
    w2t = jnp.pad(W2.T, ((0, _KP - (_H + _D)), (0, 0)))
    w3t = jnp.pad(W3.T, ((0, 0), (0, 128 - 1)))
    b0r = b0[None, :]
    b1r = b1[None, :]
    b2r = b2[None, :]
    b3r = jnp.pad(b3[None, :], ((0, 0), (0, 128 - 1)))

    grid = (_B // _R,)
    ray = lambda i: (i, 0)
    rep2 = lambda i: (0, 0)
    t_out, v_out, y_out = pl.pallas_call(
        _body,
        grid=grid,
        in_specs=[
            pl.BlockSpec((_R, 128), ray),
            pl.BlockSpec((_R, 128), ray),
            pl.BlockSpec((128, _H), rep2),
            pl.BlockSpec((1, _H), rep2),
            pl.BlockSpec((_KP, _H), rep2),
            pl.BlockSpec((1, _H), rep2),
            pl.BlockSpec((_KP, _H), rep2),
            pl.BlockSpec((1, _H), rep2),
            pl.BlockSpec((_H, 128), rep2),
            pl.BlockSpec((1, 128), rep2),
        ],
        out_specs=[
            pl.BlockSpec((_R, _K), ray),
            pl.BlockSpec((_R, _K), ray),
            pl.BlockSpec((_R, _K), ray),
        ],
        out_shape=[
            jax.ShapeDtypeStruct((_B, _K), jnp.float32),
            jax.ShapeDtypeStruct((_B, _K), jnp.float32),
            jax.ShapeDtypeStruct((_B, _K), jnp.float32),
        ],
    )(epp, spp, w0t, b0r, w1t, b1r, w2t, b2r, w3t, b3r)
    return t_out, v_out > 0, y_out[:, :, None]


# speedup vs baseline: 47.3711x; 2.6502x over previous
import jax
import jax.numpy as jnp
from jax.experimental import pallas as pl
from jax.experimental.pallas import tpu as pltpu

_B = 256
_D = 3
_H = 256
_K = 64
_EPS = 1e-06
_NC = 4
_R = 32
_KP = 384

_PD = jax.lax.Precision.DEFAULT
_PH = jax.lax.Precision.HIGHEST


def _dot(a, b, prec):
    return jax.lax.dot_general(
        a, b, (((1,), (0,)), ((), ())), precision=prec,
        preferred_element_type=jnp.float32)


def _dotb(a, b):
    return jax.lax.dot_general(
        a, b, (((2,), (1,)), ((0,), (0,))), precision=_PH,
        preferred_element_type=jnp.float32)


def _insert(t, vm, z, kout):
    R, kin = t.shape
    zL = z[:, :-1, :]
    dz = z[:, 1:, :] - zL
    segv = vm[:, :-1] * vm[:, 1:]
    dvalid = jnp.abs(dz) > _EPS
    dsafe = jnp.where(dvalid, dz, jnp.float32(1.0))
    a = -zL / dsafe
    cva = (segv[:, :, None] > 0) & dvalid & (a > _EPS) & (a < 1.0 - _EPS)
    BIGK = jnp.float32(10.0)
    key = jnp.where(cva, a, BIGK)
    lane = jax.lax.broadcasted_iota(jnp.int32, key.shape, 2)
    alphas = []
    mcnt = jnp.zeros((R, kin - 1), jnp.float32)
    for _ in range(_NC):
        m = jnp.min(key, axis=2, keepdims=True)
        is_min = key == m
        sel = jnp.min(jnp.where(is_min, lane, _H), axis=2, keepdims=True)
        key = jnp.where(lane == sel, BIGK, key)
        alphas.append(m[:, :, 0])
        mcnt = mcnt + (m[:, :, 0] < 9.0).astype(jnp.float32)

    pad1 = jnp.zeros((R, 1), jnp.float32)
    mpad = jnp.concatenate([mcnt, pad1], axis=1)
    lt = (jax.lax.broadcasted_iota(jnp.int32, (kin, kin), 0)
          < jax.lax.broadcasted_iota(jnp.int32, (kin, kin), 1)).astype(jnp.float32)
    Cm = _dot(mpad, lt, _PH)
    ivec = jax.lax.broadcasted_iota(jnp.int32, (R, kin), 1).astype(jnp.float32)
    BIGP = jnp.float32(1e9)
    p = jnp.where(vm > 0, ivec + Cm, BIGP)
    nnew = jnp.minimum(
        jnp.sum(vm, axis=1, keepdims=True) + jnp.sum(mcnt, axis=1, keepdims=True),
        jnp.float32(_K))

    jsub = jax.lax.broadcasted_iota(jnp.int32, (R, kout, 1), 1).astype(jnp.float32)
    le = (p[:, None, :] <= jsub).astype(jnp.float32)
    iLf = jnp.sum(le, axis=2, keepdims=True) - 1.0
    ilane = jax.lax.broadcasted_iota(jnp.int32, (R, kout, kin), 2).astype(jnp.float32)
    O = (ilane == iLf).astype(jnp.float32)
    posL = jnp.sum(O * p[:, None, :], axis=2, keepdims=True)
    cidx = jsub - posL - 1.0
    aout = jnp.zeros((R, kout, 1), jnp.float32)
    for c in range(_NC):
        acpad = jnp.concatenate([alphas[c], pad1], axis=1)
        gc = jnp.sum(O * acpad[:, None, :], axis=2, keepdims=True)
        aout = aout + jnp.where(cidx == jnp.float32(c), gc, jnp.float32(0.0))
    is_old = posL == jsub
    aout = jnp.where(is_old, jnp.float32(0.0), aout)

    vnew = (jsub < nnew[:, :, None]).astype(jnp.float32)
    dtfull = jnp.concatenate([t[:, 1:] - t[:, :-1], pad1], axis=1)
    tL = jnp.sum(O * t[:, None, :], axis=2, keepdims=True)
    dtv = jnp.sum(O * dtfull[:, None, :], axis=2, keepdims=True)
    tnew = jnp.where(vnew > 0, tL + aout * dtv, jnp.float32(1.0))

    dzpad = jnp.concatenate([dz, jnp.zeros((R, 1, _H), jnp.float32)], axis=1)
    zcat = jnp.concatenate([z, dzpad], axis=2)
    g = _dotb(O, zcat)
    znew = g[:, :, :_H] + aout * g[:, :, _H:]
    znew = jnp.where(vnew > 0, znew, jnp.float32(0.0))
    return tnew[:, :, 0], vnew[:, :, 0], znew


def _body(ep_ref, sp_ref, w0t_ref, b0_ref, w1t_ref, b1_ref, w2t_ref, b2_ref,
          w3t_ref, b3_ref, t_out_ref, v_out_ref, y_out_ref):
    ep = ep_ref[...]
    sp = sp_ref[...]
    d = ep - sp
    R = ep.shape[0]

    K0 = 8
    t = jnp.where(jax.lax.broadcasted_iota(jnp.int32, (R, K0), 1) == 0,
                  jnp.float32(0.0), jnp.float32(1.0))
    vm = (jax.lax.broadcasted_iota(jnp.int32, (R, K0), 1) < 2).astype(jnp.float32)

    a0 = _dot(d, w0t_ref[...], _PD)
    beta0 = _dot(sp, w0t_ref[...], _PD) + b0_ref[...]
    z = a0[:, None, :] * t[:, :, None] + beta0[:, None, :]
    z = jnp.where(vm[:, :, None] > 0, z, jnp.float32(0.0))
    t, vm, z = _insert(t, vm, z, 8)

    for wt_ref, b_ref, kout in ((w1t_ref, b1_ref, 32), (w2t_ref, b2_ref, 64)):
        kk = t.shape[1]
        h = jnp.where(vm[:, :, None] > 0, jnp.maximum(z, 0.0), jnp.float32(0.0))
        coords = sp[:, None, :_D] + t[:, :, None] * d[:, None, :_D]
        coords = jnp.where(vm[:, :, None] > 0, coords, jnp.float32(0.0))
        cpad = jnp.pad(coords, ((0, 0), (0, 0), (0, _KP - _H - _D)))
        hin = jnp.concatenate([h, cpad], axis=2)
        z = _dot(hin.reshape(R * kk, _KP), wt_ref[...], _PD) + b_ref[...]
        z = z.reshape(R, kk, _H)
        z = jnp.where(vm[:, :, None] > 0, z, jnp.float32(0.0))
        t, vm, z = _insert(t, vm, z, kout)

    h = jnp.where(vm[:, :, None] > 0, jnp.maximum(z, 0.0), jnp.float32(0.0))
    y = _dot(h.reshape(R * _K, _H), w3t_ref[...], _PD)
    y = (y + b3_ref[...]).reshape(R, _K, 128)[:, :, 0]
    y = jnp.where(vm > 0, y, jnp.float32(0.0))

    t_out_ref[...] = t
    v_out_ref[...] = vm
    y_out_ref[...] = y


def kernel(end_points, start_points, W0, b0, W1, b1, W2, b2, W3, b3):
    epp = jnp.pad(end_points, ((0, 0), (0, 128 - _D)))
    spp = jnp.pad(start_points, ((0, 0), (0, 128 - _D)))
    w0t = jnp.pad(W0.T, ((0, 128 - _D), (0, 0)))
    w1t = jnp.pad(W1.T, ((0, _KP - (_H + _D)), (0, 0)))
    w2t = jnp.pad(W2.T, ((0, _KP - (_H + _D)), (0, 0)))
    w3t = jnp.pad(W3.T, ((0, 0), (0, 128 - 1)))
    b0r = b0[None, :]
    b1r = b1[None, :]
    b2r = b2[None, :]
    b3r = jnp.pad(b3[None, :], ((0, 0), (0, 128 - 1)))

    grid = (_B // _R,)
    ray = lambda i: (i, 0)
    rep2 = lambda i: (0, 0)
    t_out, v_out, y_out = pl.pallas_call(
        _body,
        grid=grid,
        in_specs=[
            pl.BlockSpec((_R, 128), ray),
            pl.BlockSpec((_R, 128), ray),
            pl.BlockSpec((128, _H), rep2),
            pl.BlockSpec((1, _H), rep2),
            pl.BlockSpec((_KP, _H), rep2),
            pl.BlockSpec((1, _H), rep2),
            pl.BlockSpec((_KP, _H), rep2),
            pl.BlockSpec((1, _H), rep2),
            pl.BlockSpec((_H, 128), rep2),
            pl.BlockSpec((1, 128), rep2),
        ],
        out_specs=[
            pl.BlockSpec((_R, _K), ray),
            pl.BlockSpec((_R, _K), ray),
            pl.BlockSpec((_R, _K), ray),
        ],
        out_shape=[
            jax.ShapeDtypeStruct((_B, _K), jnp.float32),
            jax.ShapeDtypeStruct((_B, _K), jnp.float32),
            jax.ShapeDtypeStruct((_B, _K), jnp.float32),
        ],
    )(epp, spp, w0t, b0r, w1t, b1r, w2t, b2r, w3t, b3r)
    return t_out, v_out > 0, y_out[:, :, None]


# speedup vs baseline: 54.1025x; 1.1421x over previous
import jax
import jax.numpy as jnp
from jax.experimental import pallas as pl
from jax.experimental.pallas import tpu as pltpu

_B = 256
_D = 3
_H = 256
_K = 64
_EPS = 1e-06
_NC = 4
_R = 32
_KP = 384

_PD = jax.lax.Precision.DEFAULT
_PH = jax.lax.Precision.HIGHEST


def _dot(a, b, prec):
    return jax.lax.dot_general(
        a, b, (((1,), (0,)), ((), ())), precision=prec,
        preferred_element_type=jnp.float32)


def _dotb(a, b, prec):
    return jax.lax.dot_general(
        a, b, (((2,), (1,)), ((0,), (0,))), precision=prec,
        preferred_element_type=jnp.float32)


def _gatherb(o, x):
    xh = x.astype(jnp.bfloat16).astype(jnp.float32)
    r1 = x - xh
    xm = r1.astype(jnp.bfloat16).astype(jnp.float32)
    xl = r1 - xm
    return (_dotb(o, xh, _PD) + _dotb(o, xm, _PD)) + _dotb(o, xl, _PD)


def _insert(t, vm, z, kout):
    R, kin = t.shape
    zL = z[:, :-1, :]
    dz = z[:, 1:, :] - zL
    segv = vm[:, :-1] * vm[:, 1:]
    dvalid = jnp.abs(dz) > _EPS
    dsafe = jnp.where(dvalid, dz, jnp.float32(1.0))
    a = -zL / dsafe
    cva = (segv[:, :, None] > 0) & dvalid & (a > _EPS) & (a < 1.0 - _EPS)
    BIGK = jnp.float32(10.0)
    key = jnp.where(cva, a, BIGK)
    m1 = jnp.min(key, axis=2)
    f2 = jnp.where(key > m1[:, :, None], key, BIGK)
    m2 = jnp.min(f2, axis=2)
    f3 = jnp.where(f2 > m2[:, :, None], f2, BIGK)
    m3 = jnp.min(f3, axis=2)
    f4 = jnp.where(f3 > m3[:, :, None], f3, BIGK)
    m4 = jnp.min(f4, axis=2)
    c1 = jnp.sum((key == m1[:, :, None]).astype(jnp.float32), axis=2)
    C2 = c1 + jnp.sum((key == m2[:, :, None]).astype(jnp.float32), axis=2)
    C3 = C2 + jnp.sum((key == m3[:, :, None]).astype(jnp.float32), axis=2)
    alphas = []
    mcnt = jnp.zeros((R, kin - 1), jnp.float32)
    for c in range(_NC):
        cf = jnp.float32(c)
        sc = jnp.where(cf < c1, m1,
                       jnp.where(cf < C2, m2, jnp.where(cf < C3, m3, m4)))
        alphas.append(sc)
        mcnt = mcnt + (sc < 9.0).astype(jnp.float32)

    pad1 = jnp.zeros((R, 1), jnp.float32)
    mpad = jnp.concatenate([mcnt, pad1], axis=1)
    lt = (jax.lax.broadcasted_iota(jnp.int32, (kin, kin), 0)
          < jax.lax.broadcasted_iota(jnp.int32, (kin, kin), 1)).astype(jnp.float32)
    Cm = _dot(mpad, lt, _PD)
    ivec = jax.lax.broadcasted_iota(jnp.int32, (R, kin), 1).astype(jnp.float32)
    BIGP = jnp.float32(1e9)
    p = jnp.where(vm > 0, ivec + Cm, BIGP)
    nnew = jnp.minimum(
        jnp.sum(vm, axis=1, keepdims=True) + jnp.sum(mcnt, axis=1, keepdims=True),
        jnp.float32(_K))

    jsub = jax.lax.broadcasted_iota(jnp.int32, (R, kout, 1), 1).astype(jnp.float32)
    le = (p[:, None, :] <= jsub).astype(jnp.float32)
    iLf = jnp.sum(le, axis=2, keepdims=True) - 1.0
    ilane = jax.lax.broadcasted_iota(jnp.int32, (R, kout, kin), 2).astype(jnp.float32)
    O = (ilane == iLf).astype(jnp.float32)
    posL = jnp.sum(O * p[:, None, :], axis=2, keepdims=True)
    cidx = jsub - posL - 1.0
    aout = jnp.zeros((R, kout, 1), jnp.float32)
    for c in range(_NC):
        acpad = jnp.concatenate([alphas[c], pad1], axis=1)
        gc = jnp.sum(O * acpad[:, None, :], axis=2, keepdims=True)
        aout = aout + jnp.where(cidx == jnp.float32(c), gc, jnp.float32(0.0))
    is_old = posL == jsub
    aout = jnp.where(is_old, jnp.float32(0.0), aout)

    vnew = (jsub < nnew[:, :, None]).astype(jnp.float32)
    dtfull = jnp.concatenate([t[:, 1:] - t[:, :-1], pad1], axis=1)
    tL = jnp.sum(O * t[:, None, :], axis=2, keepdims=True)
    dtv = jnp.sum(O * dtfull[:, None, :], axis=2, keepdims=True)
    tnew = jnp.where(vnew > 0, tL + aout * dtv, jnp.float32(1.0))

    dzpad = jnp.concatenate([dz, jnp.zeros((R, 1, _H), jnp.float32)], axis=1)
    zcat = jnp.concatenate([z, dzpad], axis=2)
    g = _gatherb(O, zcat)
    znew = g[:, :, :_H] + aout * g[:, :, _H:]
    znew = jnp.where(vnew > 0, znew, jnp.float32(0.0))
    return tnew[:, :, 0], vnew[:, :, 0], znew


def _body(ep_ref, sp_ref, w0t_ref, b0_ref, w1t_ref, b1_ref, w2t_ref, b2_ref,
          w3t_ref, b3_ref, t_out_ref, v_out_ref, y_out_ref):
    ep = ep_ref[...]
    sp = sp_ref[...]
    d = ep - sp
    R = ep.shape[0]

    K0 = 8
    t = jnp.where(jax.lax.broadcasted_iota(jnp.int32, (R, K0), 1) == 0,
                  jnp.float32(0.0), jnp.float32(1.0))
    vm = (jax.lax.broadcasted_iota(jnp.int32, (R, K0), 1) < 2).astype(jnp.float32)

    a0 = _dot(d, w0t_ref[...], _PD)
    beta0 = _dot(sp, w0t_ref[...], _PD) + b0_ref[...]
    z = a0[:, None, :] * t[:, :, None] + beta0[:, None, :]
    z = jnp.where(vm[:, :, None] > 0, z, jnp.float32(0.0))
    t, vm, z = _insert(t, vm, z, 8)

    for wt_ref, b_ref, kout in ((w1t_ref, b1_ref, 32), (w2t_ref, b2_ref, 64)):
        kk = t.shape[1]
        h = jnp.where(vm[:, :, None] > 0, jnp.maximum(z, 0.0), jnp.float32(0.0))
        coords = sp[:, None, :_D] + t[:, :, None] * d[:, None, :_D]
        coords = jnp.where(vm[:, :, None] > 0, coords, jnp.float32(0.0))
        cpad = jnp.pad(coords, ((0, 0), (0, 0), (0, _KP - _H - _D)))
        hin = jnp.concatenate([h, cpad], axis=2)
        z = _dot(hin.reshape(R * kk, _KP), wt_ref[...], _PD) + b_ref[...]
        z = z.reshape(R, kk, _H)
        z = jnp.where(vm[:, :, None] > 0, z, jnp.float32(0.0))
        t, vm, z = _insert(t, vm, z, kout)

    h = jnp.where(vm[:, :, None] > 0, jnp.maximum(z, 0.0), jnp.float32(0.0))
    y = _dot(h.reshape(R * _K, _H), w3t_ref[...], _PD)
    y = (y + b3_ref[...]).reshape(R, _K, 128)[:, :, 0]
    y = jnp.where(vm > 0, y, jnp.float32(0.0))

    t_out_ref[...] = t
    v_out_ref[...] = vm
    y_out_ref[...] = y


def kernel(end_points, start_points, W0, b0, W1, b1, W2, b2, W3, b3):
    epp = jnp.pad(end_points, ((0, 0), (0, 128 - _D)))
    spp = jnp.pad(start_points, ((0, 0), (0, 128 - _D)))
    w0t = jnp.pad(W0.T, ((0, 128 - _D), (0, 0)))
    w1t = jnp.pad(W1.T, ((0, _KP - (_H + _D)), (0, 0)))
    w2t = jnp.pad(W2.T, ((0, _KP - (_H + _D)), (0, 0)))
    w3t = jnp.pad(W3.T, ((0, 0), (0, 128 - 1)))
    b0r = b0[None, :]
    b1r = b1[None, :]
    b2r = b2[None, :]
    b3r = jnp.pad(b3[None, :], ((0, 0), (0, 128 - 1)))

    grid = (_B // _R,)
    ray = lambda i: (i, 0)
    rep2 = lambda i: (0, 0)
    t_out, v_out, y_out = pl.pallas_call(
        _body,
        grid=grid,
        in_specs=[
            pl.BlockSpec((_R, 128), ray),
            pl.BlockSpec((_R, 128), ray),
            pl.BlockSpec((128, _H), rep2),
            pl.BlockSpec((1, _H), rep2),
            pl.BlockSpec((_KP, _H), rep2),
            pl.BlockSpec((1, _H), rep2),
            pl.BlockSpec((_KP, _H), rep2),
            pl.BlockSpec((1, _H), rep2),
            pl.BlockSpec((_H, 128), rep2),
            pl.BlockSpec((1, 128), rep2),
        ],
        out_specs=[
            pl.BlockSpec((_R, _K), ray),
            pl.BlockSpec((_R, _K), ray),
            pl.BlockSpec((_R, _K), ray),
        ],
        out_shape=[
            jax.ShapeDtypeStruct((_B, _K), jnp.float32),
            jax.ShapeDtypeStruct((_B, _K), jnp.float32),
            jax.ShapeDtypeStruct((_B, _K), jnp.float32),
        ],
    )(epp, spp, w0t, b0r, w1t, b1r, w2t, b2r, w3t, b3r)
    return t_out, v_out > 0, y_out[:, :, None]
